# baseline (device time: 88736 ns/iter reference)
import jax
import jax.numpy as jnp
from jax import lax
from jax.experimental import pallas as pl
from jax.experimental.pallas import tpu as pltpu

N_RING = 4


def kernel(Q, K, V):
    b, q_len, h, d = Q.shape
    k_len = K.shape[1]
    hd = h * d
    nb = b // N_RING
    scale = d ** -0.5

    mx = lax.axis_index("x")
    my = lax.axis_index("y")
    r = 2 * mx + jnp.where(mx == 1, 1 - my, my)

    Kq = lax.dynamic_slice_in_dim(K, r * nb, nb, axis=0)
    Vq = lax.dynamic_slice_in_dim(V, r * nb, nb, axis=0)
    Kf = Kq.reshape(nb, k_len, hd)
    Vf = Vq.reshape(nb, k_len, hd)
    Qf = (Q * scale).reshape(b, hd)

    def ring_coords(pos):
        x = jnp.where(pos >= 2, 1, 0)
        y = jnp.where((pos == 1) | (pos == 2), 1, 0)
        return x, y

    def body(q_ref, k_ref, v_ref, out_ref,
             acc_ref, recv_ref, comm_ref,
             z_send, z_recv, ring_send, ring_recv):
        bi = pl.program_id(0)
        mx = lax.axis_index("x")
        my = lax.axis_index("y")
        mz = lax.axis_index("z")
        partner = (mx, my, 1 - mz)
        r = 2 * mx + jnp.where(mx == 1, 1 - my, my)
        rx, ry = ring_coords(lax.rem(r + 1, 4))
        lx, ly = ring_coords(lax.rem(r + 3, 4))

        @pl.when(bi == 0)
        def _entry_barrier():
            bar = pltpu.get_barrier_semaphore()
            for dev in (partner, (lx, ly, mz), (rx, ry, mz)):
                pl.semaphore_signal(
                    bar, inc=1, device_id=dev,
                    device_id_type=pl.DeviceIdType.MESH,
                )
            pl.semaphore_wait(bar, 3)

        lane = lax.broadcasted_iota(jnp.int32, (h, hd), 1)
        sub = lax.broadcasted_iota(jnp.int32, (h, hd), 0)
        mask = (lane // d) == sub

        qrow = q_ref[pl.ds(r * nb + bi, 1), :]
        qexp = jnp.where(mask, jnp.broadcast_to(qrow, (h, hd)), 0.0)
        kb = k_ref[0]
        vb = v_ref[0]

        s = lax.dot_general(
            qexp, kb, (((1,), (1,)), ((), ())),
            precision=lax.Precision.HIGHEST,
            preferred_element_type=jnp.float32,
        )
        m = jnp.max(s, axis=1, keepdims=True)
        p = jnp.exp(s - m)
        l = jnp.sum(p, axis=1, keepdims=True)

        g = lax.dot_general(
            p, vb, (((1,), (0,)), ((), ())),
            precision=lax.Precision.HIGHEST,
            preferred_element_type=jnp.float32,
        )
        o_flat = jnp.sum(jnp.where(mask, g, 0.0), axis=0, keepdims=True)
        m_flat = jnp.sum(
            jnp.where(mask, jnp.broadcast_to(m, (h, hd)), 0.0),
            axis=0, keepdims=True)
        l_flat = jnp.sum(
            jnp.where(mask, jnp.broadcast_to(l, (h, hd)), 0.0),
            axis=0, keepdims=True)

        acc_ref[pl.ds(bi, 1), 0, :] = o_flat
        acc_ref[pl.ds(bi, 1), 1, :] = m_flat
        acc_ref[pl.ds(bi, 1), 2, :] = l_flat

        def z_rdma(i):
            return pltpu.make_async_remote_copy(
                src_ref=acc_ref.at[i], dst_ref=recv_ref.at[i],
                send_sem=z_send.at[i], recv_sem=z_recv.at[i],
                device_id=partner,
                device_id_type=pl.DeviceIdType.MESH,
            )

        z_rdma(bi).start()

        @pl.when(bi == nb - 1)
        def _combine_and_gather():
            for i in range(nb):
                z_rdma(i).wait()

            for i in range(nb):
                ma = acc_ref[i, 1, :].reshape(1, hd)
                mb = recv_ref[i, 1, :].reshape(1, hd)
                mn = jnp.maximum(ma, mb)
                alpha = jnp.exp(ma - mn)
                beta = jnp.exp(mb - mn)
                oc = (alpha * acc_ref[i, 0, :].reshape(1, hd)
                      + beta * recv_ref[i, 0, :].reshape(1, hd)) \
                    / (alpha * acc_ref[i, 2, :].reshape(1, hd)
                       + beta * recv_ref[i, 2, :].reshape(1, hd))
                comm_ref[0, i, :] = oc[0]
                out_ref[pl.ds(r, 1), i, :] = oc

            for hop in range(N_RING - 1):
                send_slot = hop % 2
                recv_slot = (hop + 1) % 2
                ring = pltpu.make_async_remote_copy(
                    src_ref=comm_ref.at[send_slot],
                    dst_ref=comm_ref.at[recv_slot],
                    send_sem=ring_send.at[send_slot],
                    recv_sem=ring_recv.at[recv_slot],
                    device_id=(rx, ry, mz),
                    device_id_type=pl.DeviceIdType.MESH,
                )
                ring.start()
                ring.wait()
                origin = lax.rem(r + 3 - hop, 4)
                out_ref[pl.ds(origin, 1), :, :] = comm_ref[recv_slot][jnp.newaxis]

    out = pl.pallas_call(
        body,
        grid=(nb,),
        out_shape=jax.ShapeDtypeStruct((N_RING, nb, hd), jnp.float32),
        in_specs=[
            pl.BlockSpec((b, hd), lambda i: (0, 0)),
            pl.BlockSpec((1, k_len, hd), lambda i: (i, 0, 0)),
            pl.BlockSpec((1, k_len, hd), lambda i: (i, 0, 0)),
        ],
        out_specs=pl.BlockSpec((N_RING, nb, hd), lambda i: (0, 0, 0)),
        scratch_shapes=[
            pltpu.VMEM((nb, 3, hd), jnp.float32),
            pltpu.VMEM((nb, 3, hd), jnp.float32),
            pltpu.VMEM((2, nb, hd), jnp.float32),
            pltpu.SemaphoreType.DMA((nb,)),
            pltpu.SemaphoreType.DMA((nb,)),
            pltpu.SemaphoreType.DMA((2,)),
            pltpu.SemaphoreType.DMA((2,)),
        ],
        compiler_params=pltpu.CompilerParams(
            dimension_semantics=("arbitrary",),
            collective_id=0,
        ),
    )(Qf, Kf, Vf)

    return out.reshape(b, q_len, h, d)


# device time: 82659 ns/iter; 1.0735x vs baseline; 1.0735x over previous
import jax
import jax.numpy as jnp
from jax import lax
from jax.experimental import pallas as pl
from jax.experimental.pallas import tpu as pltpu

N_RING = 4


def kernel(Q, K, V):
    b, q_len, h, d = Q.shape
    k_len = K.shape[1]
    hd = h * d
    nb = b // N_RING
    scale = d ** -0.5

    mx = lax.axis_index("x")
    my = lax.axis_index("y")
    r = 2 * mx + jnp.where(mx == 1, 1 - my, my)

    Kq = lax.dynamic_slice_in_dim(K, r * nb, nb, axis=0)
    Vq = lax.dynamic_slice_in_dim(V, r * nb, nb, axis=0)
    Kf = Kq.reshape(nb, k_len, hd)
    Vf = Vq.reshape(nb, k_len, hd)
    Qf = (Q * scale).reshape(b, hd)

    def ring_coords(pos):
        x = jnp.where(pos >= 2, 1, 0)
        y = jnp.where((pos == 1) | (pos == 2), 1, 0)
        return x, y

    def body(q_ref, k_ref, v_ref, out_ref,
             acc_ref, recv_ref, comm_ref,
             z_send, z_recv, ring_send, ring_recv):
        bi = pl.program_id(0)
        mx = lax.axis_index("x")
        my = lax.axis_index("y")
        mz = lax.axis_index("z")
        partner = (mx, my, 1 - mz)
        r = 2 * mx + jnp.where(mx == 1, 1 - my, my)
        rx, ry = ring_coords(lax.rem(r + 1, 4))
        lx, ly = ring_coords(lax.rem(r + 3, 4))

        @pl.when(bi == 0)
        def _entry_barrier():
            bar = pltpu.get_barrier_semaphore()
            for dev in (partner, (lx, ly, mz), (rx, ry, mz)):
                pl.semaphore_signal(
                    bar, inc=1, device_id=dev,
                    device_id_type=pl.DeviceIdType.MESH,
                )
            pl.semaphore_wait(bar, 3)

        lane = lax.broadcasted_iota(jnp.int32, (h, hd), 1)
        sub = lax.broadcasted_iota(jnp.int32, (h, hd), 0)
        mask = (lane // d) == sub

        qrow = q_ref[pl.ds(r * nb + bi, 1), :]
        qexp = jnp.where(mask, jnp.broadcast_to(qrow, (h, hd)), 0.0)
        kb = k_ref[0]
        vb = v_ref[0]

        s = lax.dot_general(
            qexp, kb, (((1,), (1,)), ((), ())),
            precision=lax.Precision.HIGHEST,
            preferred_element_type=jnp.float32,
        )
        m = jnp.max(s, axis=1, keepdims=True)
        p = jnp.exp(s - m)
        l = jnp.sum(p, axis=1, keepdims=True)

        g = lax.dot_general(
            p, vb, (((1,), (0,)), ((), ())),
            preferred_element_type=jnp.float32,
        )
        o_flat = jnp.sum(jnp.where(mask, g, 0.0), axis=0, keepdims=True)
        m_flat = jnp.sum(
            jnp.where(mask, jnp.broadcast_to(m, (h, hd)), 0.0),
            axis=0, keepdims=True)
        l_flat = jnp.sum(
            jnp.where(mask, jnp.broadcast_to(l, (h, hd)), 0.0),
            axis=0, keepdims=True)

        acc_ref[pl.ds(bi, 1), 0, :] = o_flat
        acc_ref[pl.ds(bi, 1), 1, :] = m_flat
        acc_ref[pl.ds(bi, 1), 2, :] = l_flat

        def z_rdma(i):
            return pltpu.make_async_remote_copy(
                src_ref=acc_ref.at[i], dst_ref=recv_ref.at[i],
                send_sem=z_send.at[i], recv_sem=z_recv.at[i],
                device_id=partner,
                device_id_type=pl.DeviceIdType.MESH,
            )

        z_rdma(bi).start()

        @pl.when(bi == nb - 1)
        def _combine_and_gather():
            for i in range(nb):
                z_rdma(i).wait()

            for i in range(nb):
                ma = acc_ref[i, 1, :].reshape(1, hd)
                mb = recv_ref[i, 1, :].reshape(1, hd)
                mn = jnp.maximum(ma, mb)
                alpha = jnp.exp(ma - mn)
                beta = jnp.exp(mb - mn)
                oc = (alpha * acc_ref[i, 0, :].reshape(1, hd)
                      + beta * recv_ref[i, 0, :].reshape(1, hd)) \
                    / (alpha * acc_ref[i, 2, :].reshape(1, hd)
                       + beta * recv_ref[i, 2, :].reshape(1, hd))
                comm_ref[0, i, :] = oc[0]
                out_ref[pl.ds(r, 1), i, :] = oc

            for hop in range(N_RING - 1):
                send_slot = hop % 2
                recv_slot = (hop + 1) % 2
                ring = pltpu.make_async_remote_copy(
                    src_ref=comm_ref.at[send_slot],
                    dst_ref=comm_ref.at[recv_slot],
                    send_sem=ring_send.at[send_slot],
                    recv_sem=ring_recv.at[recv_slot],
                    device_id=(rx, ry, mz),
                    device_id_type=pl.DeviceIdType.MESH,
                )
                ring.start()
                ring.wait()
                origin = lax.rem(r + 3 - hop, 4)
                out_ref[pl.ds(origin, 1), :, :] = comm_ref[recv_slot][jnp.newaxis]

    out = pl.pallas_call(
        body,
        grid=(nb,),
        out_shape=jax.ShapeDtypeStruct((N_RING, nb, hd), jnp.float32),
        in_specs=[
            pl.BlockSpec((b, hd), lambda i: (0, 0)),
            pl.BlockSpec((1, k_len, hd), lambda i: (i, 0, 0)),
            pl.BlockSpec((1, k_len, hd), lambda i: (i, 0, 0)),
        ],
        out_specs=pl.BlockSpec((N_RING, nb, hd), lambda i: (0, 0, 0)),
        scratch_shapes=[
            pltpu.VMEM((nb, 3, hd), jnp.float32),
            pltpu.VMEM((nb, 3, hd), jnp.float32),
            pltpu.VMEM((2, nb, hd), jnp.float32),
            pltpu.SemaphoreType.DMA((nb,)),
            pltpu.SemaphoreType.DMA((nb,)),
            pltpu.SemaphoreType.DMA((2,)),
            pltpu.SemaphoreType.DMA((2,)),
        ],
        compiler_params=pltpu.CompilerParams(
            dimension_semantics=("arbitrary",),
            collective_id=0,
        ),
    )(Qf, Kf, Vf)

    return out.reshape(b, q_len, h, d)
